# baseline (speedup 1.0000x reference)
import jax
import jax.numpy as jnp
from jax import lax
from jax.experimental import pallas as pl
from jax.experimental.pallas import tpu as pltpu

B, S, D = 64, 512, 2048
NUM_PROMPTS = 200
TOP_K = 64
NP_PAD = 256
BCH = 8
NCH = B // BCH
XCH = 8
NXCH = B // XCH


def _body(cls_ref, prompts_ref, wq_ref, bq_ref, wp_ref, bp_ref, x_hbm,
          out_hbm, pproj_s, sel0_s, sel1_s, sem_x, sem_sel):
    x_copies = []
    for c in range(NXCH):
        cp = pltpu.make_async_copy(
            x_hbm.at[pl.ds(c * XCH, XCH)],
            out_hbm.at[pl.ds(c * XCH, XCH), pl.ds(TOP_K, S)],
            sem_x)
        cp.start()
        x_copies.append(cp)

    prompts = prompts_ref[...]
    pproj = lax.dot_general(prompts, wp_ref[...],
                            (((1,), (1,)), ((), ())),
                            preferred_element_type=jnp.float32)
    pproj_s[0:NUM_PROMPTS, :] = pproj + bp_ref[...]
    pproj_s[NUM_PROMPTS:NP_PAD, :] = jnp.zeros(
        (NP_PAD - NUM_PROMPTS, D), jnp.float32)

    q = lax.dot_general(cls_ref[...], wq_ref[...],
                        (((1,), (1,)), ((), ())),
                        preferred_element_type=jnp.float32)
    q = q + bq_ref[...]
    qn = q * lax.rsqrt(jnp.maximum(
        jnp.sum(q * q, axis=1, keepdims=True), 1e-24))
    pn = prompts * lax.rsqrt(jnp.maximum(
        jnp.sum(prompts * prompts, axis=1, keepdims=True), 1e-24))
    sim = lax.dot_general(qn, pn, (((1,), (1,)), ((), ())),
                          preferred_element_type=jnp.float32)
    sim = jnp.concatenate(
        [sim, jnp.full((B, NP_PAD - NUM_PROMPTS), -2.0, jnp.float32)],
        axis=1)

    sel_bufs = [sel0_s, sel1_s]
    sel_copies = []
    for c in range(NCH):
        sc = sim[c * BCH:(c + 1) * BCH, :]
        s_i = sc.reshape(BCH, NP_PAD, 1)
        s_j = sc.reshape(BCH, 1, NP_PAD)
        ii = lax.broadcasted_iota(jnp.int32, (BCH, NP_PAD, NP_PAD), 1)
        jj = lax.broadcasted_iota(jnp.int32, (BCH, NP_PAD, NP_PAD), 2)
        beats = (s_j > s_i) | ((s_j == s_i) & (jj < ii))
        rank = jnp.sum(beats.astype(jnp.int32), axis=2)
        kk = lax.broadcasted_iota(jnp.int32, (BCH, TOP_K, NP_PAD), 1)
        onehot = (kk == rank.reshape(BCH, 1, NP_PAD)).astype(jnp.float32)
        sel = lax.dot_general(onehot.reshape(BCH * TOP_K, NP_PAD),
                              pproj_s[...], (((1,), (0,)), ((), ())),
                              preferred_element_type=jnp.float32)
        buf = sel_bufs[c % 2]
        if c >= 2:
            sel_copies[c - 2].wait()
        buf[...] = sel.reshape(BCH, TOP_K, D)
        cp = pltpu.make_async_copy(
            buf, out_hbm.at[pl.ds(c * BCH, BCH), pl.ds(0, TOP_K)], sem_sel)
        cp.start()
        sel_copies.append(cp)

    for cp in sel_copies[-2:]:
        cp.wait()
    for cp in x_copies:
        cp.wait()


@jax.jit
def kernel(x, prompts_embeddings, Wq, bq, Wp, bp):
    cls = x[:, 0, :]
    bq2 = bq.reshape(1, D)
    bp2 = bp.reshape(1, D)

    vmem = pl.BlockSpec(memory_space=pltpu.MemorySpace.HBM)
    out = pl.pallas_call(
        _body,
        in_specs=[
            pl.BlockSpec(memory_space=pltpu.MemorySpace.VMEM),
            pl.BlockSpec(memory_space=pltpu.MemorySpace.VMEM),
            pl.BlockSpec(memory_space=pltpu.MemorySpace.VMEM),
            pl.BlockSpec(memory_space=pltpu.MemorySpace.VMEM),
            pl.BlockSpec(memory_space=pltpu.MemorySpace.VMEM),
            pl.BlockSpec(memory_space=pltpu.MemorySpace.VMEM),
            vmem,
        ],
        out_specs=pl.BlockSpec(memory_space=pltpu.MemorySpace.HBM),
        out_shape=jax.ShapeDtypeStruct((B, TOP_K + S, D), jnp.float32),
        scratch_shapes=[
            pltpu.VMEM((NP_PAD, D), jnp.float32),
            pltpu.VMEM((BCH, TOP_K, D), jnp.float32),
            pltpu.VMEM((BCH, TOP_K, D), jnp.float32),
            pltpu.SemaphoreType.DMA,
            pltpu.SemaphoreType.DMA,
        ],
    )(cls, prompts_embeddings, Wq, bq2, Wp, bp2, x)
    return out


# speedup vs baseline: 1.0002x; 1.0002x over previous
import jax
import jax.numpy as jnp
from jax import lax
from jax.experimental import pallas as pl
from jax.experimental.pallas import tpu as pltpu

B, S, D = 64, 512, 2048
NUM_PROMPTS = 200
TOP_K = 64
NP_PAD = 256
BCH = 8
NCH = B // BCH
XCH = 1
NXCH = B // XCH


def _body(cls_ref, prompts_ref, wq_ref, bq_ref, wp_ref, bp_ref, x_hbm,
          out_hbm, pproj_s, sel0_s, sel1_s, sem_x, sem_sel):
    x_copies = []
    for c in range(NXCH):
        cp = pltpu.make_async_copy(
            x_hbm.at[pl.ds(c * XCH, XCH)],
            out_hbm.at[pl.ds(c * XCH, XCH), pl.ds(TOP_K, S)],
            sem_x)
        cp.start()
        x_copies.append(cp)

    prompts = prompts_ref[...]
    pproj = lax.dot_general(prompts, wp_ref[...],
                            (((1,), (1,)), ((), ())),
                            preferred_element_type=jnp.float32)
    pproj_s[0:NUM_PROMPTS, :] = pproj + bp_ref[...]
    pproj_s[NUM_PROMPTS:NP_PAD, :] = jnp.zeros(
        (NP_PAD - NUM_PROMPTS, D), jnp.float32)

    q = lax.dot_general(cls_ref[...], wq_ref[...],
                        (((1,), (1,)), ((), ())),
                        preferred_element_type=jnp.float32)
    q = q + bq_ref[...]
    qn = q * lax.rsqrt(jnp.maximum(
        jnp.sum(q * q, axis=1, keepdims=True), 1e-24))
    pn = prompts * lax.rsqrt(jnp.maximum(
        jnp.sum(prompts * prompts, axis=1, keepdims=True), 1e-24))
    sim = lax.dot_general(qn, pn, (((1,), (1,)), ((), ())),
                          preferred_element_type=jnp.float32)
    sim = jnp.concatenate(
        [sim, jnp.full((B, NP_PAD - NUM_PROMPTS), -2.0, jnp.float32)],
        axis=1)

    sel_bufs = [sel0_s, sel1_s]
    sel_copies = []
    for c in range(NCH):
        sc = sim[c * BCH:(c + 1) * BCH, :]
        s_i = sc.reshape(BCH, NP_PAD, 1)
        s_j = sc.reshape(BCH, 1, NP_PAD)
        ii = lax.broadcasted_iota(jnp.int32, (BCH, NP_PAD, NP_PAD), 1)
        jj = lax.broadcasted_iota(jnp.int32, (BCH, NP_PAD, NP_PAD), 2)
        beats = (s_j > s_i) | ((s_j == s_i) & (jj < ii))
        rank = jnp.sum(beats.astype(jnp.int32), axis=2)
        kk = lax.broadcasted_iota(jnp.int32, (BCH, TOP_K, NP_PAD), 1)
        onehot = (kk == rank.reshape(BCH, 1, NP_PAD)).astype(jnp.float32)
        sel = lax.dot_general(onehot.reshape(BCH * TOP_K, NP_PAD),
                              pproj_s[...], (((1,), (0,)), ((), ())),
                              preferred_element_type=jnp.float32)
        buf = sel_bufs[c % 2]
        if c >= 2:
            sel_copies[c - 2].wait()
        buf[...] = sel.reshape(BCH, TOP_K, D)
        cp = pltpu.make_async_copy(
            buf, out_hbm.at[pl.ds(c * BCH, BCH), pl.ds(0, TOP_K)], sem_sel)
        cp.start()
        sel_copies.append(cp)

    for cp in sel_copies[-2:]:
        cp.wait()
    for cp in x_copies:
        cp.wait()


@jax.jit
def kernel(x, prompts_embeddings, Wq, bq, Wp, bp):
    cls = x[:, 0, :]
    bq2 = bq.reshape(1, D)
    bp2 = bp.reshape(1, D)

    vmem = pl.BlockSpec(memory_space=pltpu.MemorySpace.HBM)
    out = pl.pallas_call(
        _body,
        in_specs=[
            pl.BlockSpec(memory_space=pltpu.MemorySpace.VMEM),
            pl.BlockSpec(memory_space=pltpu.MemorySpace.VMEM),
            pl.BlockSpec(memory_space=pltpu.MemorySpace.VMEM),
            pl.BlockSpec(memory_space=pltpu.MemorySpace.VMEM),
            pl.BlockSpec(memory_space=pltpu.MemorySpace.VMEM),
            pl.BlockSpec(memory_space=pltpu.MemorySpace.VMEM),
            vmem,
        ],
        out_specs=pl.BlockSpec(memory_space=pltpu.MemorySpace.HBM),
        out_shape=jax.ShapeDtypeStruct((B, TOP_K + S, D), jnp.float32),
        scratch_shapes=[
            pltpu.VMEM((NP_PAD, D), jnp.float32),
            pltpu.VMEM((BCH, TOP_K, D), jnp.float32),
            pltpu.VMEM((BCH, TOP_K, D), jnp.float32),
            pltpu.SemaphoreType.DMA,
            pltpu.SemaphoreType.DMA,
        ],
    )(cls, prompts_embeddings, Wq, bq2, Wp, bp2, x)
    return out


# speedup vs baseline: 35.2693x; 35.2636x over previous
import jax
import jax.numpy as jnp
from jax import lax
from jax.experimental import pallas as pl
from jax.experimental.pallas import tpu as pltpu

B, S, D = 64, 512, 2048
NUM_PROMPTS = 200
TOP_K = 64
NP_PAD = 256


def _body(cls_ref, prompts_ref, wq_ref, bq_ref, wp_ref, bp_ref, x_ref,
          out_hbm, pproj_s, sim_s, sel_s, sem_x, sem_sel):
    b = pl.program_id(0)

    @pl.when(b == 0)
    def _head():
        prompts = prompts_ref[...]
        pproj = lax.dot_general(prompts, wp_ref[...],
                                (((1,), (1,)), ((), ())),
                                preferred_element_type=jnp.float32)
        pproj_s[0:NUM_PROMPTS, :] = pproj + bp_ref[...]
        pproj_s[NUM_PROMPTS:NP_PAD, :] = jnp.zeros(
            (NP_PAD - NUM_PROMPTS, D), jnp.float32)

        q = lax.dot_general(cls_ref[...], wq_ref[...],
                            (((1,), (1,)), ((), ())),
                            preferred_element_type=jnp.float32)
        q = q + bq_ref[...]
        qn = q * lax.rsqrt(jnp.maximum(
            jnp.sum(q * q, axis=1, keepdims=True), 1e-24))
        pn = prompts * lax.rsqrt(jnp.maximum(
            jnp.sum(prompts * prompts, axis=1, keepdims=True), 1e-24))
        sim = lax.dot_general(qn, pn, (((1,), (1,)), ((), ())),
                              preferred_element_type=jnp.float32)
        sim_s[:, 0:NUM_PROMPTS] = sim
        sim_s[:, NUM_PROMPTS:NP_PAD] = jnp.full(
            (B, NP_PAD - NUM_PROMPTS), -2.0, jnp.float32)

    srow = sim_s[pl.ds(b, 1), :]
    s_i = srow.reshape(NP_PAD, 1)
    s_j = srow
    ii = lax.broadcasted_iota(jnp.int32, (NP_PAD, NP_PAD), 0)
    jj = lax.broadcasted_iota(jnp.int32, (NP_PAD, NP_PAD), 1)
    beats = (s_j > s_i) | ((s_j == s_i) & (jj < ii))
    rank = jnp.sum(beats.astype(jnp.int32), axis=1)
    kk = lax.broadcasted_iota(jnp.int32, (TOP_K, NP_PAD), 0)
    onehot = (kk == rank.reshape(1, NP_PAD)).astype(jnp.float32)
    sel_s[...] = lax.dot_general(onehot, pproj_s[...],
                                 (((1,), (0,)), ((), ())),
                                 preferred_element_type=jnp.float32)

    sel_cp = pltpu.make_async_copy(
        sel_s, out_hbm.at[b, pl.ds(0, TOP_K)], sem_sel)
    sel_cp.start()
    x_cp = pltpu.make_async_copy(
        x_ref.at[0], out_hbm.at[b, pl.ds(TOP_K, S)], sem_x)
    x_cp.start()
    x_cp.wait()
    sel_cp.wait()


@jax.jit
def kernel(x, prompts_embeddings, Wq, bq, Wp, bp):
    cls = x[:, 0, :]
    bq2 = bq.reshape(1, D)
    bp2 = bp.reshape(1, D)

    VM = pltpu.MemorySpace.VMEM
    out = pl.pallas_call(
        _body,
        grid=(B,),
        in_specs=[
            pl.BlockSpec((B, D), lambda b: (0, 0), memory_space=VM),
            pl.BlockSpec((NUM_PROMPTS, D), lambda b: (0, 0), memory_space=VM),
            pl.BlockSpec((D, D), lambda b: (0, 0), memory_space=VM),
            pl.BlockSpec((1, D), lambda b: (0, 0), memory_space=VM),
            pl.BlockSpec((D, D), lambda b: (0, 0), memory_space=VM),
            pl.BlockSpec((1, D), lambda b: (0, 0), memory_space=VM),
            pl.BlockSpec((1, S, D), lambda b: (b, 0, 0), memory_space=VM),
        ],
        out_specs=pl.BlockSpec(memory_space=pltpu.MemorySpace.HBM),
        out_shape=jax.ShapeDtypeStruct((B, TOP_K + S, D), jnp.float32),
        scratch_shapes=[
            pltpu.VMEM((NP_PAD, D), jnp.float32),
            pltpu.VMEM((B, NP_PAD), jnp.float32),
            pltpu.VMEM((TOP_K, D), jnp.float32),
            pltpu.SemaphoreType.DMA,
            pltpu.SemaphoreType.DMA,
        ],
        compiler_params=pltpu.CompilerParams(
            dimension_semantics=("arbitrary",)),
    )(cls, prompts_embeddings, Wq, bq2, Wp, bp2, x)
    return out


# speedup vs baseline: 36.1022x; 1.0236x over previous
import jax
import jax.numpy as jnp
from jax import lax
from jax.experimental import pallas as pl
from jax.experimental.pallas import tpu as pltpu

B, S, D = 64, 512, 2048
NUM_PROMPTS = 200
TOP_K = 64
NP_PAD = 256


def _body(cls_ref, prompts_ref, wq_hbm, bq_ref, wp_hbm, bp_ref, x_ref,
          out_hbm, wq_s, wp_s, pproj_s, sim_s, sel_s, sem_x, sem_sel, sem_w):
    b = pl.program_id(0)

    x_cp = pltpu.make_async_copy(
        x_ref.at[0], out_hbm.at[b, pl.ds(TOP_K, S)], sem_x)
    x_cp.start()

    @pl.when(b == 0)
    def _head():
        wq_cp = pltpu.make_async_copy(wq_hbm, wq_s, sem_w)
        wq_cp.start()
        wp_cp = pltpu.make_async_copy(wp_hbm, wp_s, sem_w)
        wp_cp.start()
        wq_cp.wait()
        wp_cp.wait()

        prompts = prompts_ref[...]
        pproj = lax.dot_general(prompts, wp_s[...],
                                (((1,), (1,)), ((), ())),
                                preferred_element_type=jnp.float32)
        pproj_s[0:NUM_PROMPTS, :] = pproj + bp_ref[...]
        pproj_s[NUM_PROMPTS:NP_PAD, :] = jnp.zeros(
            (NP_PAD - NUM_PROMPTS, D), jnp.float32)

        q = lax.dot_general(cls_ref[...], wq_s[...],
                            (((1,), (1,)), ((), ())),
                            preferred_element_type=jnp.float32)
        q = q + bq_ref[...]
        qn = q * lax.rsqrt(jnp.maximum(
            jnp.sum(q * q, axis=1, keepdims=True), 1e-24))
        pn = prompts * lax.rsqrt(jnp.maximum(
            jnp.sum(prompts * prompts, axis=1, keepdims=True), 1e-24))
        sim = lax.dot_general(qn, pn, (((1,), (1,)), ((), ())),
                              preferred_element_type=jnp.float32)
        sim_s[:, 0:NUM_PROMPTS] = sim
        sim_s[:, NUM_PROMPTS:NP_PAD] = jnp.full(
            (B, NP_PAD - NUM_PROMPTS), -2.0, jnp.float32)

    srow = sim_s[pl.ds(b, 1), :]
    s_i = srow.reshape(NP_PAD, 1)
    s_j = srow
    ii = lax.broadcasted_iota(jnp.int32, (NP_PAD, NP_PAD), 0)
    jj = lax.broadcasted_iota(jnp.int32, (NP_PAD, NP_PAD), 1)
    beats = (s_j > s_i) | ((s_j == s_i) & (jj < ii))
    rank = jnp.sum(beats.astype(jnp.int32), axis=1)
    kk = lax.broadcasted_iota(jnp.int32, (TOP_K, NP_PAD), 0)
    onehot = (kk == rank.reshape(1, NP_PAD)).astype(jnp.float32)
    sel_s[...] = lax.dot_general(onehot, pproj_s[...],
                                 (((1,), (0,)), ((), ())),
                                 preferred_element_type=jnp.float32)

    sel_cp = pltpu.make_async_copy(
        sel_s, out_hbm.at[b, pl.ds(0, TOP_K)], sem_sel)
    sel_cp.start()
    x_cp.wait()
    sel_cp.wait()


@jax.jit
def kernel(x, prompts_embeddings, Wq, bq, Wp, bp):
    cls = x[:, 0, :]
    bq2 = bq.reshape(1, D)
    bp2 = bp.reshape(1, D)

    VM = pltpu.MemorySpace.VMEM
    HB = pltpu.MemorySpace.HBM
    out = pl.pallas_call(
        _body,
        grid=(B,),
        in_specs=[
            pl.BlockSpec((B, D), lambda b: (0, 0), memory_space=VM),
            pl.BlockSpec((NUM_PROMPTS, D), lambda b: (0, 0), memory_space=VM),
            pl.BlockSpec(memory_space=HB),
            pl.BlockSpec((1, D), lambda b: (0, 0), memory_space=VM),
            pl.BlockSpec(memory_space=HB),
            pl.BlockSpec((1, D), lambda b: (0, 0), memory_space=VM),
            pl.BlockSpec((1, S, D), lambda b: (b, 0, 0), memory_space=VM),
        ],
        out_specs=pl.BlockSpec(memory_space=HB),
        out_shape=jax.ShapeDtypeStruct((B, TOP_K + S, D), jnp.float32),
        scratch_shapes=[
            pltpu.VMEM((D, D), jnp.float32),
            pltpu.VMEM((D, D), jnp.float32),
            pltpu.VMEM((NP_PAD, D), jnp.float32),
            pltpu.VMEM((B, NP_PAD), jnp.float32),
            pltpu.VMEM((TOP_K, D), jnp.float32),
            pltpu.SemaphoreType.DMA,
            pltpu.SemaphoreType.DMA,
            pltpu.SemaphoreType.DMA,
        ],
        compiler_params=pltpu.CompilerParams(
            dimension_semantics=("arbitrary",)),
    )(cls, prompts_embeddings, Wq, bq2, Wp, bp2, x)
    return out
